# baseline (speedup 1.0000x reference)
import functools

import jax
import jax.numpy as jnp
from jax import lax
from jax.experimental import pallas as pl
from jax.experimental.pallas import tpu as pltpu
from jax.experimental.pallas import tpu_sc as plsc

BATCH = 4096
SEQ_LEN = 200
EMBED_DIM = 64
VOCAB = 1000000

NC, NS = 2, 16
NW = NC * NS

NSLAB = VOCAB // 128
VTAIL = VOCAB - NSLAB * 128
NT = 246

_mesh = plsc.VectorSubcoreMesh(
    core_axis_name="c", subcore_axis_name="s", num_cores=NC, num_subcores=NS
)


@functools.partial(
    pl.kernel,
    out_type=jax.ShapeDtypeStruct((VOCAB // 2, 128), jnp.float32),
    mesh=_mesh,
    scratch_types=[
        pltpu.VMEM((2, 64, 128), jnp.float32),
        pltpu.VMEM((64, 128), jnp.float32),
        pltpu.VMEM((64, 128), jnp.float32),
        pltpu.SemaphoreType.DMA,
        pltpu.SemaphoreType.DMA,
        pltpu.SemaphoreType.DMA,
        pltpu.SemaphoreType.DMA,
    ],
    compiler_params=pltpu.CompilerParams(needs_layout_passes=False),
)
def _transpose_kernel(tabT_hbm, tail_hbm, out_hbm, slab_v, tout0, tout1,
                      isem0, isem1, osem0, osem1):
    isem = (isem0, isem1)
    osem = (osem0, osem1)
    tout = (tout0, tout1)
    wid = lax.axis_index("s") * NC + lax.axis_index("c")
    iota = lax.iota(jnp.int32, 16)

    def sid(t):
        return wid + t * NW

    def start_in(t, b):
        @pl.when(sid(t) < NSLAB)
        def _():
            pltpu.async_copy(
                tabT_hbm.at[:, pl.ds(sid(t) * 128, 128)], slab_v.at[b],
                isem[b],
            )

    def wait_in(b):
        pltpu.make_async_copy(
            tabT_hbm.at[:, pl.ds(0, 128)], slab_v.at[b], isem[b]
        ).wait()

    def transpose_slab(b):
        for j0 in range(0, 128, 16):
            rows = (iota + j0) >> 1
            colb = ((iota + j0) & 1) * 64
            for d in range(64):
                plsc.store_scatter(tout[b], [rows, colb + d],
                                   slab_v[b, d, pl.ds(j0, 16)])

    def start_out(t, b):
        pltpu.async_copy(
            tout[b], out_hbm.at[pl.ds(sid(t) * 64, 64), :], osem[b]
        )

    def wait_out(b):
        pltpu.make_async_copy(
            tout[b], out_hbm.at[pl.ds(0, 64), :], osem[b]
        ).wait()

    @pl.when(wid == NW - 1)
    def _():
        pltpu.sync_copy(tail_hbm, out_hbm.at[pl.ds(NSLAB * 64, VTAIL // 2), :])

    start_in(0, 0)
    start_in(1, 1)

    @pl.loop(0, NT // 2)
    def _(i0):
        for b in range(2):
            t = i0 * 2 + b
            valid = sid(t) < NSLAB

            @pl.when(valid)
            def _():
                wait_in(b)

            @pl.when(valid & (t >= 2))
            def _():
                wait_out(b)

            @pl.when(valid)
            def _():
                transpose_slab(b)
                start_out(t, b)

            start_in(t + 2, b)

    @pl.when(sid(NT - 2) < NSLAB)
    def _():
        wait_out(0)

    @pl.when(sid(NT - 1) < NSLAB)
    def _():
        wait_out(1)


BPW2 = BATCH // NW
IPW = SEQ_LEN * BPW2
NCH2 = SEQ_LEN


@functools.partial(
    pl.kernel,
    out_type=jax.ShapeDtypeStruct((SEQ_LEN, EMBED_DIM, BATCH), jnp.float32),
    mesh=_mesh,
    scratch_types=[
        pltpu.VMEM((IPW,), jnp.int32),
        pltpu.VMEM((IPW,), jnp.int32),
        pltpu.VMEM((2, BPW2, 128), jnp.float32),
        pltpu.VMEM((2, 64, 128), jnp.float32),
        pltpu.SemaphoreType.DMA,
        pltpu.SemaphoreType.DMA,
        pltpu.SemaphoreType.DMA,
        pltpu.SemaphoreType.DMA,
    ],
    compiler_params=pltpu.CompilerParams(needs_layout_passes=False),
)
def _gather_kernel(idx_hbm, tab_hbm, out_hbm, idx_v, idx2_v, g_v, t_v,
                   gsem0, gsem1, osem0, osem1):
    gsem = (gsem0, gsem1)
    osem = (osem0, osem1)
    wid = lax.axis_index("s") * NC + lax.axis_index("c")
    iota = lax.iota(jnp.int32, 16)

    pltpu.sync_copy(idx_hbm.at[pl.ds(wid * IPW, IPW)], idx_v)

    @pl.loop(0, IPW // 16, unroll=8)
    def _(k):
        idx2_v[pl.ds(k * 16, 16)] = idx_v[pl.ds(k * 16, 16)] >> 1

    def start_gather(c, b):
        pltpu.async_copy(
            tab_hbm.at[idx2_v.at[pl.ds(c * BPW2, BPW2)]], g_v.at[b], gsem[b]
        )

    def wait_gather(b):
        pltpu.make_async_copy(
            tab_hbm.at[idx2_v.at[pl.ds(0, BPW2)]], g_v.at[b], gsem[b]
        ).wait()

    def transpose_chunk(c, b):
        for jb in range(BPW2 // 16):
            rows = iota + jb * 16
            hb = (idx_v[pl.ds(c * BPW2 + jb * 16, 16)] & 1) * 64
            for d in range(EMBED_DIM):
                vec = plsc.load_gather(g_v.at[b], [rows, hb + d])
                t_v[b, d, pl.ds(jb * 16, 16)] = vec

    def start_out(c, b):
        pltpu.async_copy(
            t_v.at[b], out_hbm.at[c].at[:, pl.ds(wid * BPW2, BPW2)], osem[b]
        )

    def wait_out(b):
        pltpu.make_async_copy(
            t_v.at[b], out_hbm.at[0].at[:, pl.ds(0, BPW2)], osem[b]
        ).wait()

    start_gather(0, 0)

    @pl.loop(0, NCH2 // 2)
    def _(i0):
        for b in range(2):
            c = i0 * 2 + b
            wait_gather(b)

            @pl.when(c + 1 < NCH2)
            def _():
                start_gather(c + 1, 1 - b)

            @pl.when(c >= 2)
            def _():
                wait_out(b)

            transpose_chunk(c, b)
            start_out(c, b)

    wait_out(0)
    wait_out(1)


def kernel(token_ids, table):
    tabT = table.T
    tail = table[NSLAB * 128:].reshape(VTAIL // 2, 128)
    tab2 = _transpose_kernel(tabT, tail)

    ids = (token_ids.T.reshape(SEQ_LEN, NW, BPW2)
           .transpose(1, 0, 2).reshape(-1).astype(jnp.int32))
    out3 = _gather_kernel(ids, tab2)
    return out3.transpose(2, 0, 1)


# speedup vs baseline: 2.0773x; 2.0773x over previous
import functools

import jax
import jax.numpy as jnp
from jax import lax
from jax.experimental import pallas as pl
from jax.experimental.pallas import tpu as pltpu
from jax.experimental.pallas import tpu_sc as plsc

BATCH = 4096
SEQ_LEN = 200
EMBED_DIM = 64

NC = 2
NS = 16
NW = NC * NS

B = BATCH * SEQ_LEN
BPW = B // NW
CHUNK = 256
NB = 4
LAG = 2
NCHUNK = BPW // CHUNK

_mesh = plsc.VectorSubcoreMesh(
    core_axis_name="c", subcore_axis_name="s", num_cores=NC, num_subcores=NS
)


@functools.partial(
    pl.kernel,
    out_type=jax.ShapeDtypeStruct((B, EMBED_DIM), jnp.float32),
    mesh=_mesh,
    scratch_types=[
        pltpu.VMEM((BPW,), jnp.int32),
        pltpu.VMEM((NB, CHUNK, EMBED_DIM), jnp.float32),
    ]
    + [pltpu.SemaphoreType.DMA] * (2 * NB),
    compiler_params=pltpu.CompilerParams(
        use_tc_tiling_on_sc=False, skip_device_barrier=True
    ),
)
def _gather_kernel(idx_hbm, table_hbm, out_hbm, idx_v, rows_v, *sems):
    gsem = sems[:NB]
    ssem = sems[NB:]
    wid = lax.axis_index("s") * NC + lax.axis_index("c")
    base = wid * BPW

    pltpu.sync_copy(idx_hbm.at[pl.ds(base, BPW)], idx_v)

    def start_gather(g, b):
        pltpu.async_copy(
            table_hbm.at[idx_v.at[pl.ds(g * CHUNK, CHUNK)]],
            rows_v.at[b],
            gsem[b],
        )

    def store_chunk(g, b):
        pltpu.make_async_copy(table_hbm.at[idx_v.at[pl.ds(0, CHUNK)]],
                              rows_v.at[b], gsem[b]).wait()
        pltpu.async_copy(
            rows_v.at[b], out_hbm.at[pl.ds(base + g * CHUNK, CHUNK)], ssem[b]
        )

    def wait_store(b):
        pltpu.make_async_copy(
            rows_v.at[b], out_hbm.at[pl.ds(base, CHUNK)], ssem[b]
        ).wait()

    for t in range(NB):
        start_gather(t, t)
        if t >= LAG:
            store_chunk(t - LAG, t - LAG)

    @pl.loop(0, (NCHUNK - NB) // NB)
    def _(i0):
        for j in range(NB):
            t = NB + i0 * NB + j
            wait_store(j)
            start_gather(t, j)
            store_chunk(t - LAG, (j - LAG) % NB)

    for g in range(NCHUNK - LAG, NCHUNK):
        store_chunk(g, g % NB)
    for b in range(NB):
        wait_store(b)


def kernel(token_ids, table):
    flat = token_ids.reshape(-1).astype(jnp.int32)
    out = _gather_kernel(flat, table)
    return out.reshape(BATCH, SEQ_LEN, EMBED_DIM)
